# Initial kernel scaffold; baseline (speedup 1.0000x reference)
import jax
import jax.numpy as jnp
from jax.experimental import pallas as pl


def kernel(F, vertices, W1, b1, W2, b2, W3, b3):
    raise NotImplementedError("write your pallas kernel here")


# speedup vs baseline: 4.3600x; 4.3600x over previous
import functools

import jax
import jax.numpy as jnp
from jax import lax
from jax.experimental import pallas as pl
from jax.experimental.pallas import tpu as pltpu
from jax.experimental.pallas import tpu_sc as plsc

KNN = 16
ROWS = 256
SC_CORES = 2
SC_SUBCORES = 16
SC_WORKERS = SC_CORES * SC_SUBCORES
SC_G = 8


def _leaky(x):
    return jnp.where(x >= 0, x, 0.2 * x)


def _knn_tc_body(vrows_ref, vat_ref, va_ref, w1t_ref, b1_ref, w2t_ref,
                 b2_ref, w3t_ref, b3_ref, idx_ref, w_ref):
    b = pl.program_id(0)
    vb = vrows_ref[0]
    vat = vat_ref[0]
    va = va_ref[0]
    n = vat.shape[1]
    r = vb.shape[0]

    mm = jnp.dot(vb, vat, preferred_element_type=jnp.float32)
    s2a = (vat[0:1] * vat[0:1] + vat[1:2] * vat[1:2]) + vat[2:3] * vat[2:3]
    s2b = ((vb[:, 0:1] * vb[:, 0:1] + vb[:, 1:2] * vb[:, 1:2])
           + vb[:, 2:3] * vb[:, 2:3])
    dist = (-2.0 * mm + s2a) + s2b

    iota = lax.broadcasted_iota(jnp.int32, (r, n), 1)
    inf = jnp.float32(jnp.inf)

    idx_cols = []
    vks = []
    for _ in range(KNN):
        m = jnp.min(dist, axis=1, keepdims=True)
        cand = jnp.where(dist == m, iota, n)
        idxk = jnp.min(cand, axis=1, keepdims=True)
        onehot = iota == idxk
        vk = jnp.dot(onehot.astype(jnp.float32), va,
                     preferred_element_type=jnp.float32,
                     precision=lax.Precision.HIGHEST)
        dist = jnp.where(onehot, inf, dist)
        idx_cols.append(idxk)
        vks.append(vk)

    idx_blk = jnp.concatenate(idx_cols, axis=1)
    idx_ref[0] = idx_blk + b * n

    v0 = vks[0]
    feats = []
    for k in range(KNN):
        vk = vks[k]
        d = v0 - vk
        nrm = jnp.sqrt(jnp.maximum(jnp.sum(d * d, axis=1, keepdims=True),
                                   1e-12))
        feats.append(jnp.concatenate([v0, vk, d, nrm], axis=1))
    x = jnp.concatenate(feats, axis=0)

    h = _leaky(jnp.dot(x, w1t_ref[...],
                       preferred_element_type=jnp.float32) + b1_ref[...])
    h = _leaky(jnp.dot(h, w2t_ref[...],
                       preferred_element_type=jnp.float32) + b2_ref[...])
    w_all = jnp.dot(h, w3t_ref[...],
                    preferred_element_type=jnp.float32) + b3_ref[...]

    w_cols = [w_all[k * r:(k + 1) * r] for k in range(KNN)]
    w_ref[0] = jnp.concatenate(w_cols, axis=1)


def _knn_tc(vertices, w1t, b1, w2t, b2, w3t, b3):
    bsz, n, _ = vertices.shape
    vat = jnp.swapaxes(vertices, 1, 2)
    grid = (bsz, n // ROWS)
    return pl.pallas_call(
        _knn_tc_body,
        grid=grid,
        in_specs=[
            pl.BlockSpec((1, ROWS, 3), lambda b, i: (b, i, 0)),
            pl.BlockSpec((1, 3, n), lambda b, i: (b, 0, 0)),
            pl.BlockSpec((1, n, 3), lambda b, i: (b, 0, 0)),
            pl.BlockSpec((10, 10), lambda b, i: (0, 0)),
            pl.BlockSpec((1, 10), lambda b, i: (0, 0)),
            pl.BlockSpec((10, 10), lambda b, i: (0, 0)),
            pl.BlockSpec((1, 10), lambda b, i: (0, 0)),
            pl.BlockSpec((10, 1), lambda b, i: (0, 0)),
            pl.BlockSpec((1, 1), lambda b, i: (0, 0)),
        ],
        out_specs=[
            pl.BlockSpec((1, ROWS, KNN), lambda b, i: (b, i, 0)),
            pl.BlockSpec((1, ROWS, KNN), lambda b, i: (b, i, 0)),
        ],
        out_shape=[
            jax.ShapeDtypeStruct((bsz, n, KNN), jnp.int32),
            jax.ShapeDtypeStruct((bsz, n, KNN), jnp.float32),
        ],
    )(vertices, vat, vertices, w1t, b1, w2t, b2, w3t, b3)


def _gather_sc_body(f_hbm, idx_hbm, w_hbm, out_hbm, idx_v, rows_v, w_v,
                    out_v, sem):
    wid = lax.axis_index("s") * SC_CORES + lax.axis_index("c")
    bn = f_hbm.shape[0]
    rows_per_worker = bn // SC_WORKERS
    chunks = rows_per_worker // SC_G

    def row_body(g, carry):
        accs = [jnp.zeros((16,), jnp.float32) for _ in range(16)]
        wrow = w_v[g, pl.ds(0, KNN)]
        for k in range(KNN):
            wk = wrow[k]
            rrow = g * KNN + k
            for j in range(16):
                accs[j] = accs[j] + wk * rows_v[rrow, pl.ds(j * 16, 16)]
        for j in range(16):
            out_v[g, pl.ds(j * 16, 16)] = accs[j]
        return carry

    def chunk_body(c, carry):
        base = wid * rows_per_worker + c * SC_G
        pltpu.sync_copy(idx_hbm.at[pl.ds(base * KNN, SC_G * KNN)], idx_v)
        cp = pltpu.async_copy(f_hbm.at[idx_v], rows_v, sem)
        pltpu.sync_copy(w_hbm.at[pl.ds(base, SC_G)], w_v)
        cp.wait()
        lax.fori_loop(0, SC_G, row_body, 0, unroll=False)
        pltpu.sync_copy(out_v, out_hbm.at[pl.ds(base, SC_G)])
        return carry

    lax.fori_loop(0, chunks, chunk_body, 0, unroll=False)


def _gather_sc(f_flat, idx_flat, w_flat):
    bn, ch = f_flat.shape
    mesh = plsc.VectorSubcoreMesh(core_axis_name="c", subcore_axis_name="s")
    kfn = functools.partial(
        pl.kernel,
        out_type=jax.ShapeDtypeStruct((bn, ch), jnp.float32),
        mesh=mesh,
        scratch_types=[
            pltpu.VMEM((SC_G * KNN,), jnp.int32),
            pltpu.VMEM((SC_G * KNN, ch), jnp.float32),
            pltpu.VMEM((SC_G, KNN), jnp.float32),
            pltpu.VMEM((SC_G, ch), jnp.float32),
            pltpu.SemaphoreType.DMA,
        ],
    )(_gather_sc_body)
    return kfn(f_flat, idx_flat, w_flat)


def kernel(F, vertices, W1, b1, W2, b2, W3, b3):
    bsz, n, ch = F.shape
    idx, w = _knn_tc(vertices, W1.T, b1.reshape(1, 10), W2.T,
                     b2.reshape(1, 10), W3.T, b3.reshape(1, 1))
    f_flat = F.reshape(bsz * n, ch)
    idx_flat = idx.reshape(bsz * n * KNN)
    w_flat = w.reshape(bsz * n, KNN)
    out = _gather_sc(f_flat, idx_flat, w_flat)
    return out.reshape(bsz, n, ch)


# speedup vs baseline: 5.9979x; 1.3757x over previous
import functools

import jax
import jax.numpy as jnp
from jax import lax
from jax.experimental import pallas as pl
from jax.experimental.pallas import tpu as pltpu
from jax.experimental.pallas import tpu_sc as plsc

KNN = 16
DROWS = 512
MROWS = 16384
SC_CORES = 2
SC_SUBCORES = 16
SC_WORKERS = SC_CORES * SC_SUBCORES
SC_GD = 4
SC_G = 8
NFEAT = 10
CAND_CAP = 2048 + 16


def _leaky(x):
    return jnp.where(x >= 0, x, 0.2 * x)


def _dist_tc_body(vrows_ref, vat_ref, dist_ref):
    vb = vrows_ref[0]
    vat = vat_ref[0]
    mm = jnp.dot(vb, vat, preferred_element_type=jnp.float32)
    s2a = (vat[0:1] * vat[0:1] + vat[1:2] * vat[1:2]) + vat[2:3] * vat[2:3]
    s2b = ((vb[:, 0:1] * vb[:, 0:1] + vb[:, 1:2] * vb[:, 1:2])
           + vb[:, 2:3] * vb[:, 2:3])
    dist_ref[0] = (-2.0 * mm + s2a) + s2b


def _dist_tc(vertices):
    bsz, n, _ = vertices.shape
    vat = jnp.swapaxes(vertices, 1, 2)
    return pl.pallas_call(
        _dist_tc_body,
        grid=(bsz, n // DROWS),
        in_specs=[
            pl.BlockSpec((1, DROWS, 3), lambda b, i: (b, i, 0)),
            pl.BlockSpec((1, 3, n), lambda b, i: (b, 0, 0)),
        ],
        out_specs=pl.BlockSpec((1, DROWS, n), lambda b, i: (b, i, 0)),
        out_shape=jax.ShapeDtypeStruct((bsz, n, n), jnp.float32),
    )(vertices, vat)


def _topk_sc_body(dist_hbm, vtx_hbm, feats_hbm, idx_hbm,
                  drow_v, xv, yv, zv, cand_v, candi_v, fbuf_v, idxg_v, sem):
    wid = lax.axis_index("s") * SC_CORES + lax.axis_index("c")
    bn, n = dist_hbm.shape
    rows_pw = bn // SC_WORKERS
    nch = n // 16
    wpb = n // rows_pw
    b = wid // wpb
    lanes = lax.iota(jnp.int32, 16)
    inf = jnp.float32(jnp.inf)
    bigi = jnp.int32(2 ** 30)

    pltpu.sync_copy(vtx_hbm.at[b * 3 + 0], xv)
    pltpu.sync_copy(vtx_hbm.at[b * 3 + 1], yv)
    pltpu.sync_copy(vtx_hbm.at[b * 3 + 2], zv)

    def row_body(g, _):
        def amin(c, mt):
            return jnp.minimum(mt, drow_v[g, pl.ds(c * 16, 16)])
        mt = lax.fori_loop(0, nch, amin, jnp.full((16,), inf, jnp.float32))
        t = jnp.max(mt)

        def bcomp(c, off):
            v = drow_v[g, pl.ds(c * 16, 16)]
            mask = v <= t
            plsc.store_compressed(cand_v.at[pl.ds(off, 16)], v, mask=mask)
            plsc.store_compressed(candi_v.at[pl.ds(off, 16)],
                                  lanes + c * 16, mask=mask)
            cnt = plsc.all_reduce_population_count(mask)
            return off + cnt[0]
        off = lax.fori_loop(0, nch, bcomp, jnp.int32(0))
        cand_v[pl.ds(off, 16)] = jnp.full((16,), inf, jnp.float32)
        candi_v[pl.ds(off, 16)] = jnp.full((16,), bigi, jnp.int32)
        nchunks = (off + 15) // 16

        idxsel = jnp.zeros((16,), jnp.int32)
        n0 = jnp.int32(0)
        for k in range(KNN):
            def cmin(c, bv):
                return jnp.minimum(bv, cand_v[pl.ds(c * 16, 16)])
            bv = lax.fori_loop(0, nchunks, cmin,
                               jnp.full((16,), inf, jnp.float32))
            mval = jnp.min(bv)

            def cidx(c, bi):
                vv = cand_v[pl.ds(c * 16, 16)]
                ii = candi_v[pl.ds(c * 16, 16)]
                return jnp.minimum(bi, jnp.where(vv == mval, ii, bigi))
            bi = lax.fori_loop(0, nchunks, cidx,
                               jnp.full((16,), bigi, jnp.int32))
            nk = jnp.min(bi)

            def crem(c, _):
                vv = cand_v[pl.ds(c * 16, 16)]
                ii = candi_v[pl.ds(c * 16, 16)]
                cand_v[pl.ds(c * 16, 16)] = jnp.where(ii == nk, inf, vv)
                return 0
            lax.fori_loop(0, nchunks, crem, 0)

            idxsel = jnp.where(lanes == k, nk, idxsel)
            if k == 0:
                n0 = nk

        xg = plsc.load_gather(xv, [idxsel])
        yg = plsc.load_gather(yv, [idxsel])
        zg = plsc.load_gather(zv, [idxsel])
        n0v = jnp.full((16,), 0, jnp.int32) + n0
        x0 = plsc.load_gather(xv, [n0v])
        y0 = plsc.load_gather(yv, [n0v])
        z0 = plsc.load_gather(zv, [n0v])
        dx = x0 - xg
        dy = y0 - yg
        dz = z0 - zg
        ss = jnp.maximum((dx * dx + dy * dy) + dz * dz, 1e-12)
        si = lax.bitcast_convert_type(ss, jnp.int32)
        y = lax.bitcast_convert_type(
            jnp.int32(0x1FBD1DF5) + lax.shift_right_arithmetic(si, 1),
            jnp.float32)
        y = 0.5 * (y + ss / y)
        y = 0.5 * (y + ss / y)
        y = 0.5 * (y + ss / y)
        nrm = y

        fs = [x0, y0, z0, xg, yg, zg, dx, dy, dz, nrm]
        for i in range(NFEAT):
            plsc.store_scatter(fbuf_v,
                               [g * (16 * NFEAT) + lanes * NFEAT + i], fs[i])
        idxg_v[pl.ds(g * 16, 16)] = idxsel + b * n
        return 0

    def chunk_body(cc, _):
        base = wid * rows_pw + cc * SC_GD
        pltpu.sync_copy(dist_hbm.at[pl.ds(base, SC_GD)], drow_v)
        lax.fori_loop(0, SC_GD, row_body, 0)
        pltpu.sync_copy(fbuf_v,
                        feats_hbm.at[pl.ds(base * (16 * NFEAT),
                                           SC_GD * 16 * NFEAT)])
        pltpu.sync_copy(idxg_v, idx_hbm.at[pl.ds(base * 16, SC_GD * 16)])
        return 0

    lax.fori_loop(0, rows_pw // SC_GD, chunk_body, 0)


def _topk_sc(dist_flat, vtx_flat):
    bn, n = dist_flat.shape
    mesh = plsc.VectorSubcoreMesh(core_axis_name="c", subcore_axis_name="s")
    kfn = functools.partial(
        pl.kernel,
        out_type=[
            jax.ShapeDtypeStruct((bn * 16 * NFEAT,), jnp.float32),
            jax.ShapeDtypeStruct((bn * 16,), jnp.int32),
        ],
        mesh=mesh,
        compiler_params=pltpu.CompilerParams(needs_layout_passes=False),
        scratch_types=[
            pltpu.VMEM((SC_GD, n), jnp.float32),
            pltpu.VMEM((n,), jnp.float32),
            pltpu.VMEM((n,), jnp.float32),
            pltpu.VMEM((n,), jnp.float32),
            pltpu.VMEM((CAND_CAP,), jnp.float32),
            pltpu.VMEM((CAND_CAP,), jnp.int32),
            pltpu.VMEM((SC_GD * 16 * NFEAT,), jnp.float32),
            pltpu.VMEM((SC_GD * 16,), jnp.int32),
            pltpu.SemaphoreType.DMA,
        ],
    )(_topk_sc_body)
    return kfn(dist_flat, vtx_flat)


def _mlp_tc_body(x_ref, w1t_ref, b1_ref, w2t_ref, b2_ref, w3t_ref, b3_ref,
                 w_ref):
    x = x_ref[...]
    h = _leaky(jnp.dot(x, w1t_ref[...],
                       preferred_element_type=jnp.float32) + b1_ref[...])
    h = _leaky(jnp.dot(h, w2t_ref[...],
                       preferred_element_type=jnp.float32) + b2_ref[...])
    w_ref[...] = jnp.dot(h, w3t_ref[...],
                         preferred_element_type=jnp.float32) + b3_ref[...]


def _mlp_tc(x, w1t, b1, w2t, b2, w3t, b3):
    m = x.shape[0]
    return pl.pallas_call(
        _mlp_tc_body,
        grid=(m // MROWS,),
        in_specs=[
            pl.BlockSpec((MROWS, NFEAT), lambda i: (i, 0)),
            pl.BlockSpec((10, 10), lambda i: (0, 0)),
            pl.BlockSpec((1, 10), lambda i: (0, 0)),
            pl.BlockSpec((10, 10), lambda i: (0, 0)),
            pl.BlockSpec((1, 10), lambda i: (0, 0)),
            pl.BlockSpec((10, 1), lambda i: (0, 0)),
            pl.BlockSpec((1, 1), lambda i: (0, 0)),
        ],
        out_specs=pl.BlockSpec((MROWS, 1), lambda i: (i, 0)),
        out_shape=jax.ShapeDtypeStruct((m, 1), jnp.float32),
    )(x, w1t, b1, w2t, b2, w3t, b3)


def _gather_sc_body(f_hbm, idx_hbm, w_hbm, out_hbm, idx_v, rows_v, w_v,
                    out_v, sem):
    wid = lax.axis_index("s") * SC_CORES + lax.axis_index("c")
    bn = f_hbm.shape[0]
    rows_per_worker = bn // SC_WORKERS

    def row_body(g, carry):
        accs = [jnp.zeros((16,), jnp.float32) for _ in range(16)]
        wrow = w_v[g, pl.ds(0, KNN)]
        for k in range(KNN):
            wk = wrow[k]
            rrow = g * KNN + k
            for j in range(16):
                accs[j] = accs[j] + wk * rows_v[rrow, pl.ds(j * 16, 16)]
        for j in range(16):
            out_v[g, pl.ds(j * 16, 16)] = accs[j]
        return carry

    def chunk_body(c, carry):
        base = wid * rows_per_worker + c * SC_G
        pltpu.sync_copy(idx_hbm.at[pl.ds(base * KNN, SC_G * KNN)], idx_v)
        cp = pltpu.async_copy(f_hbm.at[idx_v], rows_v, sem)
        pltpu.sync_copy(w_hbm.at[pl.ds(base, SC_G)], w_v)
        cp.wait()
        lax.fori_loop(0, SC_G, row_body, 0, unroll=False)
        pltpu.sync_copy(out_v, out_hbm.at[pl.ds(base, SC_G)])
        return carry

    lax.fori_loop(0, rows_per_worker // SC_G, chunk_body, 0, unroll=False)


def _gather_sc(f_flat, idx_flat, w_flat):
    bn, ch = f_flat.shape
    mesh = plsc.VectorSubcoreMesh(core_axis_name="c", subcore_axis_name="s")
    kfn = functools.partial(
        pl.kernel,
        out_type=jax.ShapeDtypeStruct((bn, ch), jnp.float32),
        mesh=mesh,
        scratch_types=[
            pltpu.VMEM((SC_G * KNN,), jnp.int32),
            pltpu.VMEM((SC_G * KNN, ch), jnp.float32),
            pltpu.VMEM((SC_G, KNN), jnp.float32),
            pltpu.VMEM((SC_G, ch), jnp.float32),
            pltpu.SemaphoreType.DMA,
        ],
    )(_gather_sc_body)
    return kfn(f_flat, idx_flat, w_flat)


def kernel(F, vertices, W1, b1, W2, b2, W3, b3):
    bsz, n, ch = F.shape
    bn = bsz * n
    dist = _dist_tc(vertices).reshape(bn, n)
    vtx_flat = jnp.swapaxes(vertices, 1, 2).reshape(bsz * 3, n)
    feats, idx_flat = _topk_sc(dist, vtx_flat)
    w = _mlp_tc(feats.reshape(bn * KNN, NFEAT), W1.T, b1.reshape(1, 10),
                W2.T, b2.reshape(1, 10), W3.T, b3.reshape(1, 1))
    out = _gather_sc(F.reshape(bn, ch), idx_flat, w.reshape(bn, KNN))
    return out.reshape(bsz, n, ch)


# speedup vs baseline: 8.2396x; 1.3738x over previous
import functools

import jax
import jax.numpy as jnp
from jax import lax
from jax.experimental import pallas as pl
from jax.experimental.pallas import tpu as pltpu
from jax.experimental.pallas import tpu_sc as plsc

KNN = 16
DROWS = 512
MROWS = 16384
SC_CORES = 2
SC_SUBCORES = 16
SC_WORKERS = SC_CORES * SC_SUBCORES
SC_GD = 4
SC_G = 8
NFEAT = 10
CAND_CAP = 2048 + 16


def _leaky(x):
    return jnp.where(x >= 0, x, 0.2 * x)


def _dist_tc_body(vrows_ref, vat_ref, dist_ref):
    vb = vrows_ref[0]
    vat = vat_ref[0]
    mm = jnp.dot(vb, vat, preferred_element_type=jnp.float32)
    s2a = (vat[0:1] * vat[0:1] + vat[1:2] * vat[1:2]) + vat[2:3] * vat[2:3]
    s2b = ((vb[:, 0:1] * vb[:, 0:1] + vb[:, 1:2] * vb[:, 1:2])
           + vb[:, 2:3] * vb[:, 2:3])
    dist_ref[0] = (-2.0 * mm + s2a) + s2b


def _dist_tc(vertices):
    bsz, n, _ = vertices.shape
    vat = jnp.swapaxes(vertices, 1, 2)
    return pl.pallas_call(
        _dist_tc_body,
        grid=(bsz, n // DROWS),
        in_specs=[
            pl.BlockSpec((1, DROWS, 3), lambda b, i: (b, i, 0)),
            pl.BlockSpec((1, 3, n), lambda b, i: (b, 0, 0)),
        ],
        out_specs=pl.BlockSpec((1, DROWS, n), lambda b, i: (b, i, 0)),
        out_shape=jax.ShapeDtypeStruct((bsz, n, n), jnp.float32),
    )(vertices, vat)


def _topk_sc_body(dist_hbm, vtx_hbm, feats_hbm, idx_hbm,
                  drow_v, xv, yv, zv, cand_v, candi_v, fbuf_v, idxg_v, sem):
    wid = lax.axis_index("s") * SC_CORES + lax.axis_index("c")
    bn, n = dist_hbm.shape
    rows_pw = bn // SC_WORKERS
    nch = n // 16
    wpb = n // rows_pw
    b = wid // wpb
    lanes = lax.iota(jnp.int32, 16)
    inf = jnp.float32(jnp.inf)
    bigi = jnp.int32(2 ** 30)

    pltpu.sync_copy(vtx_hbm.at[b * 3 + 0], xv)
    pltpu.sync_copy(vtx_hbm.at[b * 3 + 1], yv)
    pltpu.sync_copy(vtx_hbm.at[b * 3 + 2], zv)

    UNR = 8

    def row_body(par, g, _):
        def amin(c8, mt):
            for u in range(UNR):
                mt = jnp.minimum(mt, drow_v[par, g,
                                            pl.ds(c8 * (16 * UNR) + u * 16,
                                                  16)])
            return mt
        mt = lax.fori_loop(0, nch // UNR, amin,
                           jnp.full((16,), inf, jnp.float32))
        t = jnp.max(mt)

        def bcomp(c8, off):
            for u in range(UNR):
                c = c8 * UNR + u
                v = drow_v[par, g, pl.ds(c * 16, 16)]
                mask = v <= t
                plsc.store_compressed(candi_v.at[pl.ds(off, 16)],
                                      lanes + c * 16, mask=mask)
                cnt = plsc.all_reduce_population_count(mask)
                off = off + cnt[0]
            return off
        off = lax.fori_loop(0, nch // UNR, bcomp, jnp.int32(0))
        candi_v[pl.ds(off, 16)] = jnp.zeros((16,), jnp.int32)
        nchunks = (off + 15) // 16

        gsplat = jnp.full((16,), 0, jnp.int32) + g

        def brval(c, _):
            ii = candi_v[pl.ds(c * 16, 16)]
            cand_v[pl.ds(c * 16, 16)] = plsc.load_gather(
                drow_v.at[par], [gsplat, ii])
            return 0
        lax.fori_loop(0, nchunks, brval, 0)
        cand_v[pl.ds(off, 16)] = jnp.full((16,), inf, jnp.float32)

        idxsel = jnp.zeros((16,), jnp.int32)
        n0 = jnp.int32(0)
        pv = jnp.float32(-jnp.inf)
        pi = jnp.int32(-1)
        for k in range(KNN):
            def sweep(c, st):
                bv, bi = st
                vv = cand_v[pl.ds(c * 16, 16)]
                ii = candi_v[pl.ds(c * 16, 16)]
                valid = (vv > pv) | ((vv == pv) & (ii > pi))
                vv = jnp.where(valid, vv, inf)
                ii = jnp.where(valid, ii, bigi)
                better = (vv < bv) | ((vv == bv) & (ii < bi))
                return (jnp.where(better, vv, bv), jnp.where(better, ii, bi))
            bv, bi = lax.fori_loop(
                0, nchunks, sweep,
                (jnp.full((16,), inf, jnp.float32),
                 jnp.full((16,), bigi, jnp.int32)))
            mval = jnp.min(bv)
            nk = jnp.min(jnp.where(bv == mval, bi, bigi))
            pv = mval
            pi = nk
            idxsel = jnp.where(lanes == k, nk, idxsel)
            if k == 0:
                n0 = nk

        xg = plsc.load_gather(xv, [idxsel])
        yg = plsc.load_gather(yv, [idxsel])
        zg = plsc.load_gather(zv, [idxsel])
        n0v = jnp.full((16,), 0, jnp.int32) + n0
        x0 = plsc.load_gather(xv, [n0v])
        y0 = plsc.load_gather(yv, [n0v])
        z0 = plsc.load_gather(zv, [n0v])
        dx = x0 - xg
        dy = y0 - yg
        dz = z0 - zg
        ss = jnp.maximum((dx * dx + dy * dy) + dz * dz, 1e-12)
        si = lax.bitcast_convert_type(ss, jnp.int32)
        y = lax.bitcast_convert_type(
            jnp.int32(0x1FBD1DF5) + lax.shift_right_arithmetic(si, 1),
            jnp.float32)
        y = 0.5 * (y + ss / y)
        y = 0.5 * (y + ss / y)
        y = 0.5 * (y + ss / y)
        nrm = y

        fs = [x0, y0, z0, xg, yg, zg, dx, dy, dz, nrm]
        for i in range(NFEAT):
            plsc.store_scatter(fbuf_v,
                               [g * (16 * NFEAT) + lanes * NFEAT + i], fs[i])
        idxg_v[pl.ds(g * 16, 16)] = idxsel + b * n
        return 0

    nchunk_total = rows_pw // SC_GD
    row0 = wid * rows_pw
    pltpu.async_copy(dist_hbm.at[pl.ds(row0, SC_GD)], drow_v.at[0], sem)

    def chunk_body(cc, _):
        par = lax.rem(cc, 2)
        base = row0 + cc * SC_GD
        pltpu.make_async_copy(dist_hbm.at[pl.ds(base, SC_GD)],
                              drow_v.at[par], sem).wait()

        @pl.when(cc + 1 < nchunk_total)
        def _():
            pltpu.async_copy(dist_hbm.at[pl.ds(base + SC_GD, SC_GD)],
                             drow_v.at[1 - par], sem)

        lax.fori_loop(0, SC_GD, functools.partial(row_body, par), 0)
        pltpu.sync_copy(fbuf_v,
                        feats_hbm.at[pl.ds(base * (16 * NFEAT),
                                           SC_GD * 16 * NFEAT)])
        pltpu.sync_copy(idxg_v, idx_hbm.at[pl.ds(base * 16, SC_GD * 16)])
        return 0

    lax.fori_loop(0, nchunk_total, chunk_body, 0)


def _topk_sc(dist_flat, vtx_flat):
    bn, n = dist_flat.shape
    mesh = plsc.VectorSubcoreMesh(core_axis_name="c", subcore_axis_name="s")
    kfn = functools.partial(
        pl.kernel,
        out_type=[
            jax.ShapeDtypeStruct((bn * 16 * NFEAT,), jnp.float32),
            jax.ShapeDtypeStruct((bn * 16,), jnp.int32),
        ],
        mesh=mesh,
        compiler_params=pltpu.CompilerParams(needs_layout_passes=False),
        scratch_types=[
            pltpu.VMEM((2, SC_GD, n), jnp.float32),
            pltpu.VMEM((n,), jnp.float32),
            pltpu.VMEM((n,), jnp.float32),
            pltpu.VMEM((n,), jnp.float32),
            pltpu.VMEM((CAND_CAP,), jnp.float32),
            pltpu.VMEM((CAND_CAP,), jnp.int32),
            pltpu.VMEM((SC_GD * 16 * NFEAT,), jnp.float32),
            pltpu.VMEM((SC_GD * 16,), jnp.int32),
            pltpu.SemaphoreType.DMA,
        ],
    )(_topk_sc_body)
    return kfn(dist_flat, vtx_flat)


def _mlp_tc_body(x_ref, w1t_ref, b1_ref, w2t_ref, b2_ref, w3t_ref, b3_ref,
                 w_ref):
    x = x_ref[...]
    h = _leaky(jnp.dot(x, w1t_ref[...],
                       preferred_element_type=jnp.float32) + b1_ref[...])
    h = _leaky(jnp.dot(h, w2t_ref[...],
                       preferred_element_type=jnp.float32) + b2_ref[...])
    w_ref[...] = jnp.dot(h, w3t_ref[...],
                         preferred_element_type=jnp.float32) + b3_ref[...]


def _mlp_tc(x, w1t, b1, w2t, b2, w3t, b3):
    m = x.shape[0]
    return pl.pallas_call(
        _mlp_tc_body,
        grid=(m // MROWS,),
        in_specs=[
            pl.BlockSpec((MROWS, NFEAT), lambda i: (i, 0)),
            pl.BlockSpec((10, 10), lambda i: (0, 0)),
            pl.BlockSpec((1, 10), lambda i: (0, 0)),
            pl.BlockSpec((10, 10), lambda i: (0, 0)),
            pl.BlockSpec((1, 10), lambda i: (0, 0)),
            pl.BlockSpec((10, 1), lambda i: (0, 0)),
            pl.BlockSpec((1, 1), lambda i: (0, 0)),
        ],
        out_specs=pl.BlockSpec((MROWS, 1), lambda i: (i, 0)),
        out_shape=jax.ShapeDtypeStruct((m, 1), jnp.float32),
    )(x, w1t, b1, w2t, b2, w3t, b3)


def _gather_sc_body(f_hbm, idx_hbm, w_hbm, out_hbm, idx_v, rows_v, w_v,
                    out_v, sem):
    wid = lax.axis_index("s") * SC_CORES + lax.axis_index("c")
    bn = f_hbm.shape[0]
    rows_per_worker = bn // SC_WORKERS

    def row_body(g, carry):
        accs = [jnp.zeros((16,), jnp.float32) for _ in range(16)]
        wrow = w_v[g, pl.ds(0, KNN)]
        for k in range(KNN):
            wk = wrow[k]
            rrow = g * KNN + k
            for j in range(16):
                accs[j] = accs[j] + wk * rows_v[rrow, pl.ds(j * 16, 16)]
        for j in range(16):
            out_v[g, pl.ds(j * 16, 16)] = accs[j]
        return carry

    def chunk_body(c, carry):
        base = wid * rows_per_worker + c * SC_G
        pltpu.sync_copy(idx_hbm.at[pl.ds(base * KNN, SC_G * KNN)], idx_v)
        cp = pltpu.async_copy(f_hbm.at[idx_v], rows_v, sem)
        pltpu.sync_copy(w_hbm.at[pl.ds(base, SC_G)], w_v)
        cp.wait()
        lax.fori_loop(0, SC_G, row_body, 0, unroll=False)
        pltpu.sync_copy(out_v, out_hbm.at[pl.ds(base, SC_G)])
        return carry

    lax.fori_loop(0, rows_per_worker // SC_G, chunk_body, 0, unroll=False)


def _gather_sc(f_flat, idx_flat, w_flat):
    bn, ch = f_flat.shape
    mesh = plsc.VectorSubcoreMesh(core_axis_name="c", subcore_axis_name="s")
    kfn = functools.partial(
        pl.kernel,
        out_type=jax.ShapeDtypeStruct((bn, ch), jnp.float32),
        mesh=mesh,
        scratch_types=[
            pltpu.VMEM((SC_G * KNN,), jnp.int32),
            pltpu.VMEM((SC_G * KNN, ch), jnp.float32),
            pltpu.VMEM((SC_G, KNN), jnp.float32),
            pltpu.VMEM((SC_G, ch), jnp.float32),
            pltpu.SemaphoreType.DMA,
        ],
    )(_gather_sc_body)
    return kfn(f_flat, idx_flat, w_flat)


def kernel(F, vertices, W1, b1, W2, b2, W3, b3):
    bsz, n, ch = F.shape
    bn = bsz * n
    dist = _dist_tc(vertices).reshape(bn, n)
    vtx_flat = jnp.swapaxes(vertices, 1, 2).reshape(bsz * 3, n)
    feats, idx_flat = _topk_sc(dist, vtx_flat)
    w = _mlp_tc(feats.reshape(bn * KNN, NFEAT), W1.T, b1.reshape(1, 10),
                W2.T, b2.reshape(1, 10), W3.T, b3.reshape(1, 1))
    out = _gather_sc(F.reshape(bn, ch), idx_flat, w.reshape(bn, KNN))
    return out.reshape(bsz, n, ch)


# speedup vs baseline: 8.3334x; 1.0114x over previous
import functools

import jax
import jax.numpy as jnp
from jax import lax
from jax.experimental import pallas as pl
from jax.experimental.pallas import tpu as pltpu
from jax.experimental.pallas import tpu_sc as plsc

KNN = 16
DROWS = 512
MROWS = 16384
SC_CORES = 2
SC_SUBCORES = 16
SC_WORKERS = SC_CORES * SC_SUBCORES
SC_GD = 4
SC_G = 8
NFEAT = 10
CAND_CAP = 2048 + 16


def _leaky(x):
    return jnp.where(x >= 0, x, 0.2 * x)


def _dist_tc_body(vrows_ref, vat_ref, dist_ref):
    vb = vrows_ref[0]
    vat = vat_ref[0]
    mm = jnp.dot(vb, vat, preferred_element_type=jnp.float32)
    s2a = (vat[0:1] * vat[0:1] + vat[1:2] * vat[1:2]) + vat[2:3] * vat[2:3]
    s2b = ((vb[:, 0:1] * vb[:, 0:1] + vb[:, 1:2] * vb[:, 1:2])
           + vb[:, 2:3] * vb[:, 2:3])
    dist_ref[0] = (-2.0 * mm + s2a) + s2b


def _dist_tc(vertices):
    bsz, n, _ = vertices.shape
    vat = jnp.swapaxes(vertices, 1, 2)
    return pl.pallas_call(
        _dist_tc_body,
        grid=(bsz, n // DROWS),
        in_specs=[
            pl.BlockSpec((1, DROWS, 3), lambda b, i: (b, i, 0)),
            pl.BlockSpec((1, 3, n), lambda b, i: (b, 0, 0)),
        ],
        out_specs=pl.BlockSpec((1, DROWS, n), lambda b, i: (b, i, 0)),
        out_shape=jax.ShapeDtypeStruct((bsz, n, n), jnp.float32),
    )(vertices, vat)


def _topk_sc_body(dist_hbm, vtx_hbm, feats_hbm, idx_hbm,
                  drow_v, xv, yv, zv, cand_v, candi_v, fbuf_v, idxg_v, sem,
                  semf, semi):
    wid = lax.axis_index("s") * SC_CORES + lax.axis_index("c")
    bn, n = dist_hbm.shape
    rows_pw = bn // SC_WORKERS
    nch = n // 16
    wpb = n // rows_pw
    b = wid // wpb
    lanes = lax.iota(jnp.int32, 16)
    inf = jnp.float32(jnp.inf)
    bigi = jnp.int32(2 ** 30)

    pltpu.sync_copy(vtx_hbm.at[b * 3 + 0], xv)
    pltpu.sync_copy(vtx_hbm.at[b * 3 + 1], yv)
    pltpu.sync_copy(vtx_hbm.at[b * 3 + 2], zv)

    UNR = 8

    def row_body(par, g, _):
        def amin(c8, mt):
            for u in range(UNR):
                mt = jnp.minimum(mt, drow_v[par, g,
                                            pl.ds(c8 * (16 * UNR) + u * 16,
                                                  16)])
            return mt
        mt = lax.fori_loop(0, nch // UNR, amin,
                           jnp.full((16,), inf, jnp.float32))
        t = jnp.max(mt)

        def bcomp(c8, off):
            for u in range(UNR):
                c = c8 * UNR + u
                v = drow_v[par, g, pl.ds(c * 16, 16)]
                mask = v <= t
                plsc.store_compressed(candi_v.at[pl.ds(off, 16)],
                                      lanes + c * 16, mask=mask)
                cnt = plsc.all_reduce_population_count(mask)
                off = off + cnt[0]
            return off
        off = lax.fori_loop(0, nch // UNR, bcomp, jnp.int32(0))
        candi_v[pl.ds(off, 16)] = jnp.zeros((16,), jnp.int32)
        nchunks = (off + 15) // 16

        gsplat = jnp.full((16,), 0, jnp.int32) + g

        def brval(c, _):
            ii = candi_v[pl.ds(c * 16, 16)]
            cand_v[pl.ds(c * 16, 16)] = plsc.load_gather(
                drow_v.at[par], [gsplat, ii])
            return 0
        lax.fori_loop(0, nchunks, brval, 0)
        cand_v[pl.ds(off, 16)] = jnp.full((16,), inf, jnp.float32)

        idxsel = jnp.zeros((16,), jnp.int32)
        n0 = jnp.int32(0)
        pv = jnp.float32(-jnp.inf)
        pi = jnp.int32(-1)
        for k in range(KNN):
            def sweep(c, st):
                bv, bi = st
                vv = cand_v[pl.ds(c * 16, 16)]
                ii = candi_v[pl.ds(c * 16, 16)]
                valid = (vv > pv) | ((vv == pv) & (ii > pi))
                vv = jnp.where(valid, vv, inf)
                ii = jnp.where(valid, ii, bigi)
                better = (vv < bv) | ((vv == bv) & (ii < bi))
                return (jnp.where(better, vv, bv), jnp.where(better, ii, bi))
            bv, bi = lax.fori_loop(
                0, nchunks, sweep,
                (jnp.full((16,), inf, jnp.float32),
                 jnp.full((16,), bigi, jnp.int32)))
            mval = jnp.min(bv)
            nk = jnp.min(jnp.where(bv == mval, bi, bigi))
            pv = mval
            pi = nk
            idxsel = jnp.where(lanes == k, nk, idxsel)
            if k == 0:
                n0 = nk

        xg = plsc.load_gather(xv, [idxsel])
        yg = plsc.load_gather(yv, [idxsel])
        zg = plsc.load_gather(zv, [idxsel])
        n0v = jnp.full((16,), 0, jnp.int32) + n0
        x0 = plsc.load_gather(xv, [n0v])
        y0 = plsc.load_gather(yv, [n0v])
        z0 = plsc.load_gather(zv, [n0v])
        dx = x0 - xg
        dy = y0 - yg
        dz = z0 - zg
        ss = jnp.maximum((dx * dx + dy * dy) + dz * dz, 1e-12)
        si = lax.bitcast_convert_type(ss, jnp.int32)
        y = lax.bitcast_convert_type(
            jnp.int32(0x1FBD1DF5) + lax.shift_right_arithmetic(si, 1),
            jnp.float32)
        y = 0.5 * (y + ss / y)
        y = 0.5 * (y + ss / y)
        y = 0.5 * (y + ss / y)
        nrm = y

        fs = [x0, y0, z0, xg, yg, zg, dx, dy, dz, nrm]
        for i in range(NFEAT):
            plsc.store_scatter(
                fbuf_v,
                [(par * SC_GD + g) * (16 * NFEAT) + lanes * NFEAT + i], fs[i])
        idxg_v[pl.ds((par * SC_GD + g) * 16, 16)] = idxsel + b * n
        return 0

    nchunk_total = rows_pw // SC_GD
    fsz = SC_GD * 16 * NFEAT
    isz = SC_GD * 16
    row0 = wid * rows_pw
    pltpu.async_copy(dist_hbm.at[pl.ds(row0, SC_GD)], drow_v.at[0], sem)

    def chunk_body(cc, _):
        par = lax.rem(cc, 2)
        base = row0 + cc * SC_GD
        pltpu.make_async_copy(dist_hbm.at[pl.ds(base, SC_GD)],
                              drow_v.at[par], sem).wait()

        @pl.when(cc + 1 < nchunk_total)
        def _():
            pltpu.async_copy(dist_hbm.at[pl.ds(base + SC_GD, SC_GD)],
                             drow_v.at[1 - par], sem)

        @pl.when(cc >= 2)
        def _():
            b2 = (base - 2 * SC_GD)
            pltpu.make_async_copy(
                fbuf_v.at[pl.ds(par * fsz, fsz)],
                feats_hbm.at[pl.ds(b2 * (16 * NFEAT), fsz)], semf).wait()
            pltpu.make_async_copy(
                idxg_v.at[pl.ds(par * isz, isz)],
                idx_hbm.at[pl.ds(b2 * 16, isz)], semi).wait()

        lax.fori_loop(0, SC_GD, functools.partial(row_body, par), 0)
        pltpu.async_copy(fbuf_v.at[pl.ds(par * fsz, fsz)],
                         feats_hbm.at[pl.ds(base * (16 * NFEAT), fsz)], semf)
        pltpu.async_copy(idxg_v.at[pl.ds(par * isz, isz)],
                         idx_hbm.at[pl.ds(base * 16, isz)], semi)
        return 0

    lax.fori_loop(0, nchunk_total, chunk_body, 0)
    for cc in (nchunk_total - 2, nchunk_total - 1):
        par = cc % 2
        base = row0 + cc * SC_GD
        pltpu.make_async_copy(
            fbuf_v.at[pl.ds(par * fsz, fsz)],
            feats_hbm.at[pl.ds(base * (16 * NFEAT), fsz)], semf).wait()
        pltpu.make_async_copy(
            idxg_v.at[pl.ds(par * isz, isz)],
            idx_hbm.at[pl.ds(base * 16, isz)], semi).wait()


def _topk_sc(dist_flat, vtx_flat):
    bn, n = dist_flat.shape
    mesh = plsc.VectorSubcoreMesh(core_axis_name="c", subcore_axis_name="s")
    kfn = functools.partial(
        pl.kernel,
        out_type=[
            jax.ShapeDtypeStruct((bn * 16 * NFEAT,), jnp.float32),
            jax.ShapeDtypeStruct((bn * 16,), jnp.int32),
        ],
        mesh=mesh,
        compiler_params=pltpu.CompilerParams(needs_layout_passes=False),
        scratch_types=[
            pltpu.VMEM((2, SC_GD, n), jnp.float32),
            pltpu.VMEM((n,), jnp.float32),
            pltpu.VMEM((n,), jnp.float32),
            pltpu.VMEM((n,), jnp.float32),
            pltpu.VMEM((CAND_CAP,), jnp.float32),
            pltpu.VMEM((CAND_CAP,), jnp.int32),
            pltpu.VMEM((2 * SC_GD * 16 * NFEAT,), jnp.float32),
            pltpu.VMEM((2 * SC_GD * 16,), jnp.int32),
            pltpu.SemaphoreType.DMA,
            pltpu.SemaphoreType.DMA,
            pltpu.SemaphoreType.DMA,
        ],
    )(_topk_sc_body)
    return kfn(dist_flat, vtx_flat)


def _mlp_tc_body(x_ref, w1t_ref, b1_ref, w2t_ref, b2_ref, w3t_ref, b3_ref,
                 w_ref):
    x = x_ref[...]
    h = _leaky(jnp.dot(x, w1t_ref[...],
                       preferred_element_type=jnp.float32) + b1_ref[...])
    h = _leaky(jnp.dot(h, w2t_ref[...],
                       preferred_element_type=jnp.float32) + b2_ref[...])
    w_ref[...] = jnp.dot(h, w3t_ref[...],
                         preferred_element_type=jnp.float32) + b3_ref[...]


def _mlp_tc(x, w1t, b1, w2t, b2, w3t, b3):
    m = x.shape[0]
    return pl.pallas_call(
        _mlp_tc_body,
        grid=(m // MROWS,),
        in_specs=[
            pl.BlockSpec((MROWS, NFEAT), lambda i: (i, 0)),
            pl.BlockSpec((10, 10), lambda i: (0, 0)),
            pl.BlockSpec((1, 10), lambda i: (0, 0)),
            pl.BlockSpec((10, 10), lambda i: (0, 0)),
            pl.BlockSpec((1, 10), lambda i: (0, 0)),
            pl.BlockSpec((10, 1), lambda i: (0, 0)),
            pl.BlockSpec((1, 1), lambda i: (0, 0)),
        ],
        out_specs=pl.BlockSpec((MROWS, 1), lambda i: (i, 0)),
        out_shape=jax.ShapeDtypeStruct((m, 1), jnp.float32),
    )(x, w1t, b1, w2t, b2, w3t, b3)


def _gather_sc_body(f_hbm, idx_hbm, w_hbm, out_hbm, idx_v, rows_v, w_v,
                    out_v, sem):
    wid = lax.axis_index("s") * SC_CORES + lax.axis_index("c")
    bn = f_hbm.shape[0]
    rows_per_worker = bn // SC_WORKERS

    def row_body(g, carry):
        accs = [jnp.zeros((16,), jnp.float32) for _ in range(16)]
        wrow = w_v[g, pl.ds(0, KNN)]
        for k in range(KNN):
            wk = wrow[k]
            rrow = g * KNN + k
            for j in range(16):
                accs[j] = accs[j] + wk * rows_v[rrow, pl.ds(j * 16, 16)]
        for j in range(16):
            out_v[g, pl.ds(j * 16, 16)] = accs[j]
        return carry

    def chunk_body(c, carry):
        base = wid * rows_per_worker + c * SC_G
        pltpu.sync_copy(idx_hbm.at[pl.ds(base * KNN, SC_G * KNN)], idx_v)
        cp = pltpu.async_copy(f_hbm.at[idx_v], rows_v, sem)
        pltpu.sync_copy(w_hbm.at[pl.ds(base, SC_G)], w_v)
        cp.wait()
        lax.fori_loop(0, SC_G, row_body, 0, unroll=False)
        pltpu.sync_copy(out_v, out_hbm.at[pl.ds(base, SC_G)])
        return carry

    lax.fori_loop(0, rows_per_worker // SC_G, chunk_body, 0, unroll=False)


def _gather_sc(f_flat, idx_flat, w_flat):
    bn, ch = f_flat.shape
    mesh = plsc.VectorSubcoreMesh(core_axis_name="c", subcore_axis_name="s")
    kfn = functools.partial(
        pl.kernel,
        out_type=jax.ShapeDtypeStruct((bn, ch), jnp.float32),
        mesh=mesh,
        scratch_types=[
            pltpu.VMEM((SC_G * KNN,), jnp.int32),
            pltpu.VMEM((SC_G * KNN, ch), jnp.float32),
            pltpu.VMEM((SC_G, KNN), jnp.float32),
            pltpu.VMEM((SC_G, ch), jnp.float32),
            pltpu.SemaphoreType.DMA,
        ],
    )(_gather_sc_body)
    return kfn(f_flat, idx_flat, w_flat)


def kernel(F, vertices, W1, b1, W2, b2, W3, b3):
    bsz, n, ch = F.shape
    bn = bsz * n
    dist = _dist_tc(vertices).reshape(bn, n)
    vtx_flat = jnp.swapaxes(vertices, 1, 2).reshape(bsz * 3, n)
    feats, idx_flat = _topk_sc(dist, vtx_flat)
    w = _mlp_tc(feats.reshape(bn * KNN, NFEAT), W1.T, b1.reshape(1, 10),
                W2.T, b2.reshape(1, 10), W3.T, b3.reshape(1, 1))
    out = _gather_sc(F.reshape(bn, ch), idx_flat, w.reshape(bn, KNN))
    return out.reshape(bsz, n, ch)


# speedup vs baseline: 8.8506x; 1.0621x over previous
import functools

import jax
import jax.numpy as jnp
from jax import lax
from jax.experimental import pallas as pl
from jax.experimental.pallas import tpu as pltpu
from jax.experimental.pallas import tpu_sc as plsc

KNN = 16
DROWS = 512
MROWS = 16384
SC_CORES = 2
SC_SUBCORES = 16
SC_WORKERS = SC_CORES * SC_SUBCORES
SC_GD = 4
SC_G = 8
NFEAT = 10
CAND_CAP = 2048 + 16


def _leaky(x):
    return jnp.where(x >= 0, x, 0.2 * x)


def _dist_tc_body(vrows_ref, vat_ref, dist_ref):
    vb = vrows_ref[0]
    vat = vat_ref[0]
    mm = jnp.dot(vb, vat, preferred_element_type=jnp.float32)
    s2a = (vat[0:1] * vat[0:1] + vat[1:2] * vat[1:2]) + vat[2:3] * vat[2:3]
    s2b = ((vb[:, 0:1] * vb[:, 0:1] + vb[:, 1:2] * vb[:, 1:2])
           + vb[:, 2:3] * vb[:, 2:3])
    dist_ref[0] = (-2.0 * mm + s2a) + s2b


def _dist_tc(vertices):
    bsz, n, _ = vertices.shape
    vat = jnp.swapaxes(vertices, 1, 2)
    return pl.pallas_call(
        _dist_tc_body,
        grid=(bsz, n // DROWS),
        in_specs=[
            pl.BlockSpec((1, DROWS, 3), lambda b, i: (b, i, 0)),
            pl.BlockSpec((1, 3, n), lambda b, i: (b, 0, 0)),
        ],
        out_specs=pl.BlockSpec((1, DROWS, n), lambda b, i: (b, i, 0)),
        out_shape=jax.ShapeDtypeStruct((bsz, n, n), jnp.float32),
    )(vertices, vat)


def _topk_sc_body(dist_hbm, vtx_hbm, feats_hbm, idx_hbm,
                  drow_v, xv, yv, zv, cand_v, candi_v, fbuf_v, idxg_v, sem,
                  semf, semi):
    wid = lax.axis_index("s") * SC_CORES + lax.axis_index("c")
    bn, n = dist_hbm.shape
    rows_pw = bn // SC_WORKERS
    nch = n // 16
    wpb = n // rows_pw
    b = wid // wpb
    lanes = lax.iota(jnp.int32, 16)
    inf = jnp.float32(jnp.inf)
    bigi = jnp.int32(2 ** 30)

    pltpu.sync_copy(vtx_hbm.at[b * 3 + 0], xv)
    pltpu.sync_copy(vtx_hbm.at[b * 3 + 1], yv)
    pltpu.sync_copy(vtx_hbm.at[b * 3 + 2], zv)

    UNR = 8

    def row_body(par, g, _):
        def amin(c8, mt):
            for u in range(UNR):
                mt = jnp.minimum(mt, drow_v[par, g,
                                            pl.ds(c8 * (16 * UNR) + u * 16,
                                                  16)])
            return mt
        mt = lax.fori_loop(0, nch // UNR, amin,
                           jnp.full((16,), inf, jnp.float32))
        t = jnp.max(mt)

        def bcomp(c8, off):
            for u in range(UNR):
                c = c8 * UNR + u
                v = drow_v[par, g, pl.ds(c * 16, 16)]
                mask = v <= t
                plsc.store_compressed(candi_v.at[pl.ds(off, 16)],
                                      lanes + c * 16, mask=mask)
                cnt = plsc.all_reduce_population_count(mask)
                off = off + cnt[0]
            return off
        off = lax.fori_loop(0, nch // UNR, bcomp, jnp.int32(0))
        candi_v[pl.ds(off, 16)] = jnp.zeros((16,), jnp.int32)
        nchunks = (off + 15) // 16

        gsplat = jnp.full((16,), 0, jnp.int32) + g

        def brval(c, _):
            ii = candi_v[pl.ds(c * 16, 16)]
            cand_v[pl.ds(c * 16, 16)] = plsc.load_gather(
                drow_v.at[par], [gsplat, ii])
            return 0
        lax.fori_loop(0, nchunks, brval, 0)
        cand_v[pl.ds(off, 16)] = jnp.full((16,), inf, jnp.float32)

        idxsel = jnp.zeros((16,), jnp.int32)
        n0 = jnp.int32(0)
        pv = jnp.float32(-jnp.inf)
        pi = jnp.int32(-1)
        for k in range(KNN):
            def sweep(c, st):
                bv, bi = st
                vv = cand_v[pl.ds(c * 16, 16)]
                ii = candi_v[pl.ds(c * 16, 16)]
                valid = (vv > pv) | ((vv == pv) & (ii > pi))
                vv = jnp.where(valid, vv, inf)
                ii = jnp.where(valid, ii, bigi)
                better = (vv < bv) | ((vv == bv) & (ii < bi))
                return (jnp.where(better, vv, bv), jnp.where(better, ii, bi))
            bv, bi = lax.fori_loop(
                0, nchunks, sweep,
                (jnp.full((16,), inf, jnp.float32),
                 jnp.full((16,), bigi, jnp.int32)))
            mval = jnp.min(bv)
            nk = jnp.min(jnp.where(bv == mval, bi, bigi))
            pv = mval
            pi = nk
            idxsel = jnp.where(lanes == k, nk, idxsel)
            if k == 0:
                n0 = nk

        xg = plsc.load_gather(xv, [idxsel])
        yg = plsc.load_gather(yv, [idxsel])
        zg = plsc.load_gather(zv, [idxsel])
        n0v = jnp.full((16,), 0, jnp.int32) + n0
        x0 = plsc.load_gather(xv, [n0v])
        y0 = plsc.load_gather(yv, [n0v])
        z0 = plsc.load_gather(zv, [n0v])
        dx = x0 - xg
        dy = y0 - yg
        dz = z0 - zg
        ss = jnp.maximum((dx * dx + dy * dy) + dz * dz, 1e-12)
        si = lax.bitcast_convert_type(ss, jnp.int32)
        y = lax.bitcast_convert_type(
            jnp.int32(0x1FBD1DF5) + lax.shift_right_arithmetic(si, 1),
            jnp.float32)
        y = 0.5 * (y + ss / y)
        y = 0.5 * (y + ss / y)
        y = 0.5 * (y + ss / y)
        nrm = y

        fs = [x0, y0, z0, xg, yg, zg, dx, dy, dz, nrm]
        for i in range(NFEAT):
            plsc.store_scatter(
                fbuf_v,
                [(par * SC_GD + g) * (16 * NFEAT) + lanes * NFEAT + i], fs[i])
        idxg_v[pl.ds((par * SC_GD + g) * 16, 16)] = idxsel + b * n
        return 0

    nchunk_total = rows_pw // SC_GD
    fsz = SC_GD * 16 * NFEAT
    isz = SC_GD * 16
    row0 = wid * rows_pw
    pltpu.async_copy(dist_hbm.at[pl.ds(row0, SC_GD)], drow_v.at[0], sem)

    def chunk_body(cc, _):
        par = lax.rem(cc, 2)
        base = row0 + cc * SC_GD
        pltpu.make_async_copy(dist_hbm.at[pl.ds(base, SC_GD)],
                              drow_v.at[par], sem).wait()

        @pl.when(cc + 1 < nchunk_total)
        def _():
            pltpu.async_copy(dist_hbm.at[pl.ds(base + SC_GD, SC_GD)],
                             drow_v.at[1 - par], sem)

        @pl.when(cc >= 2)
        def _():
            b2 = (base - 2 * SC_GD)
            pltpu.make_async_copy(
                fbuf_v.at[pl.ds(par * fsz, fsz)],
                feats_hbm.at[pl.ds(b2 * (16 * NFEAT), fsz)], semf).wait()
            pltpu.make_async_copy(
                idxg_v.at[pl.ds(par * isz, isz)],
                idx_hbm.at[pl.ds(b2 * 16, isz)], semi).wait()

        lax.fori_loop(0, SC_GD, functools.partial(row_body, par), 0)
        pltpu.async_copy(fbuf_v.at[pl.ds(par * fsz, fsz)],
                         feats_hbm.at[pl.ds(base * (16 * NFEAT), fsz)], semf)
        pltpu.async_copy(idxg_v.at[pl.ds(par * isz, isz)],
                         idx_hbm.at[pl.ds(base * 16, isz)], semi)
        return 0

    lax.fori_loop(0, nchunk_total, chunk_body, 0)
    for cc in (nchunk_total - 2, nchunk_total - 1):
        par = cc % 2
        base = row0 + cc * SC_GD
        pltpu.make_async_copy(
            fbuf_v.at[pl.ds(par * fsz, fsz)],
            feats_hbm.at[pl.ds(base * (16 * NFEAT), fsz)], semf).wait()
        pltpu.make_async_copy(
            idxg_v.at[pl.ds(par * isz, isz)],
            idx_hbm.at[pl.ds(base * 16, isz)], semi).wait()


def _topk_sc(dist_flat, vtx_flat):
    bn, n = dist_flat.shape
    mesh = plsc.VectorSubcoreMesh(core_axis_name="c", subcore_axis_name="s")
    kfn = functools.partial(
        pl.kernel,
        out_type=[
            jax.ShapeDtypeStruct((bn * 16 * NFEAT,), jnp.float32),
            jax.ShapeDtypeStruct((bn * 16,), jnp.int32),
        ],
        mesh=mesh,
        compiler_params=pltpu.CompilerParams(needs_layout_passes=False),
        scratch_types=[
            pltpu.VMEM((2, SC_GD, n), jnp.float32),
            pltpu.VMEM((n,), jnp.float32),
            pltpu.VMEM((n,), jnp.float32),
            pltpu.VMEM((n,), jnp.float32),
            pltpu.VMEM((CAND_CAP,), jnp.float32),
            pltpu.VMEM((CAND_CAP,), jnp.int32),
            pltpu.VMEM((2 * SC_GD * 16 * NFEAT,), jnp.float32),
            pltpu.VMEM((2 * SC_GD * 16,), jnp.int32),
            pltpu.SemaphoreType.DMA,
            pltpu.SemaphoreType.DMA,
            pltpu.SemaphoreType.DMA,
        ],
    )(_topk_sc_body)
    return kfn(dist_flat, vtx_flat)


def _mlp_tc_body(x_ref, w1t_ref, b1_ref, w2t_ref, b2_ref, w3t_ref, b3_ref,
                 w_ref):
    x = x_ref[...]
    h = _leaky(jnp.dot(x, w1t_ref[...],
                       preferred_element_type=jnp.float32) + b1_ref[...])
    h = _leaky(jnp.dot(h, w2t_ref[...],
                       preferred_element_type=jnp.float32) + b2_ref[...])
    w_ref[...] = jnp.dot(h, w3t_ref[...],
                         preferred_element_type=jnp.float32) + b3_ref[...]


def _mlp_tc(x, w1t, b1, w2t, b2, w3t, b3):
    m = x.shape[0]
    return pl.pallas_call(
        _mlp_tc_body,
        grid=(m // MROWS,),
        in_specs=[
            pl.BlockSpec((MROWS, NFEAT), lambda i: (i, 0)),
            pl.BlockSpec((10, 10), lambda i: (0, 0)),
            pl.BlockSpec((1, 10), lambda i: (0, 0)),
            pl.BlockSpec((10, 10), lambda i: (0, 0)),
            pl.BlockSpec((1, 10), lambda i: (0, 0)),
            pl.BlockSpec((10, 1), lambda i: (0, 0)),
            pl.BlockSpec((1, 1), lambda i: (0, 0)),
        ],
        out_specs=pl.BlockSpec((MROWS, 1), lambda i: (i, 0)),
        out_shape=jax.ShapeDtypeStruct((m, 1), jnp.float32),
    )(x, w1t, b1, w2t, b2, w3t, b3)


def _gather_sc_body(f_hbm, idx_hbm, w_hbm, out_hbm, idx_v, rows_v, w_v,
                    out_v, semi, semg, semo):
    wid = lax.axis_index("s") * SC_CORES + lax.axis_index("c")
    bn = f_hbm.shape[0]
    rows_per_worker = bn // SC_WORKERS
    nchunk = rows_per_worker // SC_G
    row0 = wid * rows_per_worker

    def stage(c, par):
        base = row0 + c * SC_G
        pltpu.sync_copy(idx_hbm.at[pl.ds(base * KNN, SC_G * KNN)],
                        idx_v.at[par])
        pltpu.async_copy(f_hbm.at[idx_v.at[par]], rows_v.at[par], semg)
        pltpu.sync_copy(w_hbm.at[pl.ds(base, SC_G)], w_v.at[par])

    def row_body(par, g, carry):
        accs = [jnp.zeros((16,), jnp.float32) for _ in range(16)]
        wrow = w_v[par, g, pl.ds(0, KNN)]
        for k in range(KNN):
            wk = wrow[k]
            rrow = g * KNN + k
            for j in range(16):
                accs[j] = accs[j] + wk * rows_v[par, rrow, pl.ds(j * 16, 16)]
        for j in range(16):
            out_v[par, g, pl.ds(j * 16, 16)] = accs[j]
        return carry

    stage(0, 0)

    def chunk_body(c, carry):
        par = lax.rem(c, 2)
        base = row0 + c * SC_G
        pltpu.make_async_copy(f_hbm.at[idx_v.at[par]], rows_v.at[par],
                              semg).wait()

        @pl.when(c + 1 < nchunk)
        def _():
            stage(c + 1, 1 - par)

        @pl.when(c >= 2)
        def _():
            pltpu.make_async_copy(
                out_v.at[par],
                out_hbm.at[pl.ds(base - 2 * SC_G, SC_G)], semo).wait()

        lax.fori_loop(0, SC_G, functools.partial(row_body, par), 0,
                      unroll=False)
        pltpu.async_copy(out_v.at[par],
                         out_hbm.at[pl.ds(base, SC_G)], semo)
        return carry

    lax.fori_loop(0, nchunk, chunk_body, 0, unroll=False)
    for c in (nchunk - 2, nchunk - 1):
        pltpu.make_async_copy(
            out_v.at[c % 2],
            out_hbm.at[pl.ds(row0 + c * SC_G, SC_G)], semo).wait()


def _gather_sc(f_flat, idx_flat, w_flat):
    bn, ch = f_flat.shape
    mesh = plsc.VectorSubcoreMesh(core_axis_name="c", subcore_axis_name="s")
    kfn = functools.partial(
        pl.kernel,
        out_type=jax.ShapeDtypeStruct((bn, ch), jnp.float32),
        mesh=mesh,
        scratch_types=[
            pltpu.VMEM((2, SC_G * KNN), jnp.int32),
            pltpu.VMEM((2, SC_G * KNN, ch), jnp.float32),
            pltpu.VMEM((2, SC_G, KNN), jnp.float32),
            pltpu.VMEM((2, SC_G, ch), jnp.float32),
            pltpu.SemaphoreType.DMA,
            pltpu.SemaphoreType.DMA,
            pltpu.SemaphoreType.DMA,
        ],
    )(_gather_sc_body)
    return kfn(f_flat, idx_flat, w_flat)


def kernel(F, vertices, W1, b1, W2, b2, W3, b3):
    bsz, n, ch = F.shape
    bn = bsz * n
    dist = _dist_tc(vertices).reshape(bn, n)
    vtx_flat = jnp.swapaxes(vertices, 1, 2).reshape(bsz * 3, n)
    feats, idx_flat = _topk_sc(dist, vtx_flat)
    w = _mlp_tc(feats.reshape(bn * KNN, NFEAT), W1.T, b1.reshape(1, 10),
                W2.T, b2.reshape(1, 10), W3.T, b3.reshape(1, 1))
    out = _gather_sc(F.reshape(bn, ch), idx_flat, w.reshape(bn, KNN))
    return out.reshape(bsz, n, ch)
